# baseline (speedup 1.0000x reference)
import functools

import numpy as np
import jax
import jax.numpy as jnp
from jax import lax
from jax.experimental import pallas as pl
from jax.experimental.pallas import tpu as pltpu
from jax.experimental.pallas import tpu_sc as plsc

_WEIGHT = 1.0
_MARGIN = 0.1
_NUM_PAIRS = 50
_C = 96
_NSUB = 16
_NW = 32
_PAIRS_PER_W = 2

_P1 = np.array([
    173933, 223648, 132014, 66009, 197883, 225421, 230159, 148398,
    233270, 47702, 64901, 204988, 18362, 178022, 19505, 166658,
    232575, 131497, 157352, 100368, 151762, 120949, 64679, 40237,
    26140, 137874, 111796, 100120, 210019, 192558, 99161, 133721,
    199814, 218787, 217947, 40996, 157393, 29906, 62185, 146729,
    17852, 194362, 209776, 40202, 100599, 57200, 139469, 81170,
    235795, 55505], dtype=np.int32)
_P2 = np.array([
    188769, 206227, 128429, 61903, 192779, 223840, 223472, 135580,
    229723, 42630, 51107, 185528, 32687, 185195, 27192, 159481,
    243845, 138180, 169579, 96308, 163054, 126569, 77995, 43871,
    24660, 120978, 126085, 99086, 214661, 180270, 91472, 126565,
    193164, 220833, 215882, 42051, 172265, 40169, 49898, 156437,
    28108, 190785, 223097, 44297, 105234, 47930, 143064, 77125,
    234768, 68803], dtype=np.int32)

_PTS = np.zeros((_NW, 16), dtype=np.int32)
for _w in range(_NW):
    for _j in range(_PAIRS_PER_W):
        _i = _w * _PAIRS_PER_W + _j
        if _i < _NUM_PAIRS:
            _PTS[_w, 1 + 2 * _j] = _P1[_i]
            _PTS[_w, 2 + 2 * _j] = _P2[_i]

_PY = (_PTS[:, 1:5] >> 9).astype(np.int32)
_TXB = (_PTS[:, 1:5] & 511 & ~127).astype(np.int32)


@functools.lru_cache(maxsize=1)
def _build_sc_loss():
    mesh = plsc.VectorSubcoreMesh(core_axis_name="c", subcore_axis_name="s")
    return functools.partial(
        pl.kernel,
        out_type=jax.ShapeDtypeStruct((2, 16), jnp.float32),
        mesh=mesh,
        scratch_types=[
            pltpu.VMEM((16,), jnp.int32),
            pltpu.VMEM((5, _C, 128), jnp.float32),
            pltpu.VMEM((5, 128), jnp.float32),
            pltpu.VMEM((128,), jnp.float32),
            pltpu.VMEM((16, 128), jnp.float32),
            pltpu.VMEM((16,), jnp.float32),
            pltpu.VMEM_SHARED((16, 128), jnp.float32),
            pltpu.SemaphoreType.DMA,
            pltpu.SemaphoreType.DMA,
        ],
        compiler_params=pltpu.CompilerParams(
            needs_layout_passes=False, use_tc_tiling_on_sc=True),
    )(_sc_loss_body)


def _sc_loss_body(depth_hbm, targ_hbm, pts_hbm, out_hbm,
                  pts_v, dtiles_v, trows_v, pbuf_v, sbuf_v,
                  out_v, shared, sem_d, sem_t):
    core = lax.axis_index("c")
    sub = lax.axis_index("s")
    lanes = lax.iota(jnp.int32, 16)
    wid = core * _NSUB + sub

    pltpu.sync_copy(pts_hbm.at[wid], pts_v)
    for c_ in range(2):
        for t in range(_NSUB):
            w = c_ * _NSUB + t

            @pl.when(jnp.logical_and(core == c_, sub == t))
            def _(w=w):
                cps = []
                nslots = 2 * max(0, min(_PAIRS_PER_W,
                                        _NUM_PAIRS - _PAIRS_PER_W * w))
                for k in range(nslots):
                    cps.append(pltpu.async_copy(
                        depth_hbm.at[pl.ds(int(_PY[w, k]), 1),
                                     pl.ds(0, _C),
                                     pl.ds(int(_TXB[w, k]), 128)],
                        dtiles_v.at[pl.ds(k + 1, 1)], sem_d))
                    cps.append(pltpu.async_copy(
                        targ_hbm.at[pl.ds(int(_PY[w, k]), 1),
                                    pl.ds(int(_TXB[w, k]), 128)],
                        trows_v.at[pl.ds(k + 1, 1)], sem_t))
                for cp in cps:
                    cp.wait()

    part = jnp.float32(0.0)
    for j in range(_PAIRS_PER_W):
        rs = []
        ts = []
        for k in (1 + 2 * j, 2 + 2 * j):
            kv = jnp.full((16,), k, jnp.int32)
            pk = plsc.load_gather(pts_v, [kv])
            xk = lax.bitwise_and(pk, 127)
            acc = jnp.zeros((16,), jnp.float32)
            for c6 in range(_C // 16):
                acc = acc + plsc.load_gather(
                    dtiles_v, [kv, lanes + 16 * c6, xk])
            rs.append(jnp.sum(acc) * (1.0 / _C))
            ts.append(jnp.max(plsc.load_gather(trows_v, [kv, xk])))
        valid = (wid * _PAIRS_PER_W + j) < _NUM_PAIRS
        term = jnp.maximum(
            -jnp.sign(ts[1] - ts[0]) * (rs[0] - rs[1]) + _MARGIN, 0.0)
        part = part + jnp.where(valid, term, 0.0)

    for z in range(8):
        pbuf_v[pl.ds(16 * z, 16)] = (
            jnp.where(lanes == 0, part, 0.0) if z == 0
            else jnp.zeros((16,), jnp.float32))
    pltpu.sync_copy(pbuf_v, shared.at[sub])

    plsc.subcore_barrier()

    @pl.when(sub == 0)
    def _():
        pltpu.sync_copy(shared, sbuf_v)
        acc = jnp.zeros((16,), jnp.float32)
        for i in range(_NSUB):
            acc = acc + sbuf_v[i, pl.ds(0, 16)]
        part_loss = jnp.sum(acc) * (_WEIGHT / _NUM_PAIRS)
        out_v[...] = jnp.full((16,), part_loss, jnp.float32)
        pltpu.sync_copy(out_v, out_hbm.at[core])


def kernel(train_depth, train_target_depth):
    depth_t = jnp.transpose(train_depth, (0, 1, 3, 2)).reshape(512, _C, 512)
    targ2d = train_target_depth.reshape(512, 512)
    out = _build_sc_loss()(depth_t, targ2d, jnp.asarray(_PTS))
    return out[0, 0] + out[1, 0]


# speedup vs baseline: 1.0842x; 1.0842x over previous
import functools

import numpy as np
import jax
import jax.numpy as jnp
from jax import lax
from jax.experimental import pallas as pl
from jax.experimental.pallas import tpu as pltpu
from jax.experimental.pallas import tpu_sc as plsc

_WEIGHT = 1.0
_MARGIN = 0.1
_NUM_PAIRS = 50
_C = 96
_NSUB = 16
_PAIRS_PER_SUB = 4

_P1 = np.array([
    173933, 223648, 132014, 66009, 197883, 225421, 230159, 148398,
    233270, 47702, 64901, 204988, 18362, 178022, 19505, 166658,
    232575, 131497, 157352, 100368, 151762, 120949, 64679, 40237,
    26140, 137874, 111796, 100120, 210019, 192558, 99161, 133721,
    199814, 218787, 217947, 40996, 157393, 29906, 62185, 146729,
    17852, 194362, 209776, 40202, 100599, 57200, 139469, 81170,
    235795, 55505], dtype=np.int32)
_P2 = np.array([
    188769, 206227, 128429, 61903, 192779, 223840, 223472, 135580,
    229723, 42630, 51107, 185528, 32687, 185195, 27192, 159481,
    243845, 138180, 169579, 96308, 163054, 126569, 77995, 43871,
    24660, 120978, 126085, 99086, 214661, 180270, 91472, 126565,
    193164, 220833, 215882, 42051, 172265, 40169, 49898, 156437,
    28108, 190785, 223097, 44297, 105234, 47930, 143064, 77125,
    234768, 68803], dtype=np.int32)

_PTS = np.zeros((_NSUB, 16), dtype=np.int32)
for _t in range(_NSUB):
    for _j in range(_PAIRS_PER_SUB):
        _i = _t * _PAIRS_PER_SUB + _j
        if _i < _NUM_PAIRS:
            _PTS[_t, 1 + 2 * _j] = _P1[_i]
            _PTS[_t, 2 + 2 * _j] = _P2[_i]
_PY = (_PTS[:, 1:9] >> 9).astype(np.int32)
_TXB = (_PTS[:, 1:9] & 511 & ~127).astype(np.int32)
_XK = (_PTS & 127).astype(np.int32)


@functools.lru_cache(maxsize=1)
def _build_sc_loss():
    mesh = plsc.VectorSubcoreMesh(
        core_axis_name="c", subcore_axis_name="s", num_cores=1)
    return functools.partial(
        pl.kernel,
        out_type=jax.ShapeDtypeStruct((16,), jnp.float32),
        mesh=mesh,
        scratch_types=[
            pltpu.VMEM((16,), jnp.int32),
            pltpu.VMEM((9, _C, 128), jnp.float32),
            pltpu.VMEM((9, 128), jnp.float32),
            pltpu.VMEM((128,), jnp.float32),
            pltpu.VMEM((16, 128), jnp.float32),
            pltpu.VMEM((16,), jnp.float32),
            pltpu.VMEM_SHARED((16, 128), jnp.float32),
            pltpu.SemaphoreType.DMA,
            pltpu.SemaphoreType.DMA,
        ],
        compiler_params=pltpu.CompilerParams(
            needs_layout_passes=False, use_tc_tiling_on_sc=True),
    )(_sc_loss_body)


def _sc_loss_body(depth_hbm, targ_hbm, out_hbm,
                  xk_v, dtiles_v, trows_v, pbuf_v, sbuf_v,
                  out_v, shared, sem_d, sem_t):
    core = lax.axis_index("c")
    sub = lax.axis_index("s")
    lanes = lax.iota(jnp.int32, 16)

    @pl.when(core == 0)
    def _():
        for t in range(_NSUB):
            @pl.when(sub == t)
            def _(t=t):
                v = jnp.zeros((16,), jnp.int32)
                for k in range(1, 9):
                    v = jnp.where(lanes == k, int(_XK[t, k]), v)
                xk_v[...] = v
                cps = []
                nslots = 2 * max(0, min(_PAIRS_PER_SUB,
                                        _NUM_PAIRS - _PAIRS_PER_SUB * t))
                for k in range(nslots):
                    cps.append(pltpu.async_copy(
                        depth_hbm.at[pl.ds(int(_PY[t, k]), 1),
                                     pl.ds(0, _C),
                                     pl.ds(int(_TXB[t, k]), 128)],
                        dtiles_v.at[pl.ds(k + 1, 1)], sem_d))
                    cps.append(pltpu.async_copy(
                        targ_hbm.at[pl.ds(int(_PY[t, k]), 1),
                                    pl.ds(int(_TXB[t, k]), 128)],
                        trows_v.at[pl.ds(k + 1, 1)], sem_t))
                for cp in cps:
                    cp.wait()

        part = jnp.float32(0.0)
        xkv = xk_v[...]
        for j in range(_PAIRS_PER_SUB):
            rs = []
            ts = []
            for k in (1 + 2 * j, 2 + 2 * j):
                kv = jnp.full((16,), k, jnp.int32)
                xk = jnp.full(
                    (16,), jnp.sum(jnp.where(lanes == k, xkv, 0)), jnp.int32)
                acc = jnp.zeros((16,), jnp.float32)
                for c6 in range(_C // 16):
                    acc = acc + plsc.load_gather(
                        dtiles_v, [kv, lanes + 16 * c6, xk])
                rs.append(jnp.sum(acc) * (1.0 / _C))
                ts.append(jnp.max(plsc.load_gather(trows_v, [kv, xk])))
            valid = (sub * _PAIRS_PER_SUB + j) < _NUM_PAIRS
            term = jnp.maximum(
                -jnp.sign(ts[1] - ts[0]) * (rs[0] - rs[1]) + _MARGIN, 0.0)
            part = part + jnp.where(valid, term, 0.0)

        for z in range(8):
            pbuf_v[pl.ds(16 * z, 16)] = (
                jnp.where(lanes == 0, part, 0.0) if z == 0
                else jnp.zeros((16,), jnp.float32))
        pltpu.sync_copy(pbuf_v, shared.at[sub])

    plsc.subcore_barrier()

    @pl.when(jnp.logical_and(core == 0, sub == 0))
    def _():
        pltpu.sync_copy(shared, sbuf_v)
        acc = jnp.zeros((16,), jnp.float32)
        for i in range(_NSUB):
            acc = acc + sbuf_v[i, pl.ds(0, 16)]
        loss = jnp.sum(acc) * (_WEIGHT / _NUM_PAIRS)
        out_v[...] = jnp.full((16,), loss, jnp.float32)
        pltpu.sync_copy(out_v, out_hbm)


def kernel(train_depth, train_target_depth):
    depth_t = jnp.transpose(train_depth, (0, 1, 3, 2)).reshape(512, _C, 512)
    targ2d = train_target_depth.reshape(512, 512)
    out = _build_sc_loss()(depth_t, targ2d)
    return out[0]


# speedup vs baseline: 1.1095x; 1.0234x over previous
import functools

import numpy as np
import jax
import jax.numpy as jnp
from jax import lax
from jax.experimental import pallas as pl
from jax.experimental.pallas import tpu as pltpu
from jax.experimental.pallas import tpu_sc as plsc

_WEIGHT = 1.0
_MARGIN = 0.1
_NUM_PAIRS = 50
_C = 96
_NSUB = 16
_PAIRS_PER_SUB = 4

_P1 = np.array([
    173933, 223648, 132014, 66009, 197883, 225421, 230159, 148398,
    233270, 47702, 64901, 204988, 18362, 178022, 19505, 166658,
    232575, 131497, 157352, 100368, 151762, 120949, 64679, 40237,
    26140, 137874, 111796, 100120, 210019, 192558, 99161, 133721,
    199814, 218787, 217947, 40996, 157393, 29906, 62185, 146729,
    17852, 194362, 209776, 40202, 100599, 57200, 139469, 81170,
    235795, 55505], dtype=np.int32)
_P2 = np.array([
    188769, 206227, 128429, 61903, 192779, 223840, 223472, 135580,
    229723, 42630, 51107, 185528, 32687, 185195, 27192, 159481,
    243845, 138180, 169579, 96308, 163054, 126569, 77995, 43871,
    24660, 120978, 126085, 99086, 214661, 180270, 91472, 126565,
    193164, 220833, 215882, 42051, 172265, 40169, 49898, 156437,
    28108, 190785, 223097, 44297, 105234, 47930, 143064, 77125,
    234768, 68803], dtype=np.int32)

_PTS = np.zeros((_NSUB, 16), dtype=np.int32)
for _t in range(_NSUB):
    for _j in range(_PAIRS_PER_SUB):
        _i = _t * _PAIRS_PER_SUB + _j
        if _i < _NUM_PAIRS:
            _PTS[_t, 1 + 2 * _j] = _P1[_i]
            _PTS[_t, 2 + 2 * _j] = _P2[_i]
_PY = (_PTS[:, 1:9] >> 9).astype(np.int32)
_TXB = (_PTS[:, 1:9] & 511 & ~127).astype(np.int32)


@functools.lru_cache(maxsize=1)
def _build_sc_loss():
    mesh = plsc.VectorSubcoreMesh(
        core_axis_name="c", subcore_axis_name="s", num_cores=1)
    return functools.partial(
        pl.kernel,
        out_type=jax.ShapeDtypeStruct((16,), jnp.float32),
        mesh=mesh,
        scratch_types=[
            pltpu.VMEM((16,), jnp.int32),
            pltpu.VMEM((9, _C, 128), jnp.float32),
            pltpu.VMEM((9, 128), jnp.float32),
            pltpu.VMEM((128,), jnp.float32),
            pltpu.VMEM((16, 128), jnp.float32),
            pltpu.VMEM((16,), jnp.float32),
            pltpu.VMEM_SHARED((16, 128), jnp.float32),
            pltpu.SemaphoreType.DMA,
            pltpu.SemaphoreType.DMA,
        ],
        compiler_params=pltpu.CompilerParams(
            needs_layout_passes=False, use_tc_tiling_on_sc=True),
    )(_sc_loss_body)


def _sc_loss_body(depth_hbm, targ_hbm, pts_hbm, out_hbm,
                  pts_v, dtiles_v, trows_v, pbuf_v, sbuf_v,
                  out_v, shared, sem_d, sem_t):
    core = lax.axis_index("c")
    sub = lax.axis_index("s")
    lanes = lax.iota(jnp.int32, 16)

    @pl.when(core == 0)
    def _():
        pltpu.sync_copy(pts_hbm.at[sub], pts_v)
        for t in range(_NSUB):
            @pl.when(sub == t)
            def _(t=t):
                cps = []
                nslots = 2 * max(0, min(_PAIRS_PER_SUB,
                                        _NUM_PAIRS - _PAIRS_PER_SUB * t))
                for k in range(nslots):
                    cps.append(pltpu.async_copy(
                        depth_hbm.at[pl.ds(int(_PY[t, k]), 1),
                                     pl.ds(0, _C),
                                     pl.ds(int(_TXB[t, k]), 128)],
                        dtiles_v.at[pl.ds(k + 1, 1)], sem_d))
                    cps.append(pltpu.async_copy(
                        targ_hbm.at[pl.ds(int(_PY[t, k]), 1),
                                    pl.ds(int(_TXB[t, k]), 128)],
                        trows_v.at[pl.ds(k + 1, 1)], sem_t))
                for cp in cps:
                    cp.wait()

        part = jnp.float32(0.0)
        for j in range(_PAIRS_PER_SUB):
            rs = []
            ts = []
            for k in (1 + 2 * j, 2 + 2 * j):
                kv = jnp.full((16,), k, jnp.int32)
                pk = plsc.load_gather(pts_v, [kv])
                xk = lax.bitwise_and(pk, 127)
                acc = jnp.zeros((16,), jnp.float32)
                for c6 in range(_C // 16):
                    acc = acc + plsc.load_gather(
                        dtiles_v, [kv, lanes + 16 * c6, xk])
                rs.append(jnp.sum(acc) * (1.0 / _C))
                ts.append(jnp.max(
                    plsc.load_gather(trows_v, [kv, lax.bitwise_and(pk, 127)])))
            valid = (sub * _PAIRS_PER_SUB + j) < _NUM_PAIRS
            term = jnp.maximum(
                -jnp.sign(ts[1] - ts[0]) * (rs[0] - rs[1]) + _MARGIN, 0.0)
            part = part + jnp.where(valid, term, 0.0)

        for z in range(8):
            pbuf_v[pl.ds(16 * z, 16)] = (
                jnp.where(lanes == 0, part, 0.0) if z == 0
                else jnp.zeros((16,), jnp.float32))
        pltpu.sync_copy(pbuf_v, shared.at[sub])

    plsc.subcore_barrier()

    @pl.when(jnp.logical_and(core == 0, sub == 0))
    def _():
        pltpu.sync_copy(shared, sbuf_v)
        acc = jnp.zeros((16,), jnp.float32)
        for i in range(_NSUB):
            acc = acc + sbuf_v[i, pl.ds(0, 16)]
        loss = jnp.sum(acc) * (_WEIGHT / _NUM_PAIRS)
        out_v[...] = jnp.full((16,), loss, jnp.float32)
        pltpu.sync_copy(out_v, out_hbm)


def kernel(train_depth, train_target_depth):
    depth_t = jnp.transpose(train_depth, (0, 1, 3, 2)).reshape(512, _C, 512)
    targ2d = train_target_depth.reshape(512, 512)
    out = _build_sc_loss()(depth_t, targ2d, jnp.asarray(_PTS))
    return out[0]
